# baseline (speedup 1.0000x reference)
import functools

import jax
import jax.numpy as jnp
from jax import lax
from jax.experimental import pallas as pl
from jax.experimental.pallas import tpu as pltpu
from jax.experimental.pallas import tpu_sc as plsc

BATCH = 16384
DIM = 32
HPAD = 64
EPS = 1e-3

_NC = 2
_NS = 16
_NW = _NC * _NS
_BPW = BATCH // _NW
_CHUNK = 128
_NCHUNK = _BPW // _CHUNK


def _sc_gather(user_idx2, item_idx2, user_table, item_table):
    mesh = plsc.VectorSubcoreMesh(core_axis_name="c", subcore_axis_name="s")

    @functools.partial(
        pl.kernel,
        mesh=mesh,
        compiler_params=pltpu.CompilerParams(use_tc_tiling_on_sc=False),
        out_type=[
            jax.ShapeDtypeStruct((BATCH, DIM), jnp.float32),
            jax.ShapeDtypeStruct((BATCH, DIM), jnp.float32),
        ],
        scratch_types=[
            pltpu.VMEM((_NCHUNK, _CHUNK), jnp.int32),
            pltpu.VMEM((_NCHUNK, _CHUNK), jnp.int32),
            pltpu.VMEM((_BPW, DIM), jnp.float32),
            pltpu.VMEM((_BPW, DIM), jnp.float32),
            pltpu.SemaphoreType.DMA,
            pltpu.SemaphoreType.DMA,
        ],
    )
    def gather_kernel(uidx_hbm, iidx_hbm, utab_hbm, itab_hbm,
                      uout_hbm, iout_hbm,
                      uidx_v, iidx_v, urows_v, irows_v, usem, isem):
        wid = lax.axis_index("s") * _NC + lax.axis_index("c")
        row0 = wid * _NCHUNK
        base = wid * _BPW
        pltpu.sync_copy(uidx_hbm.at[pl.ds(row0, _NCHUNK)], uidx_v)
        pltpu.sync_copy(iidx_hbm.at[pl.ds(row0, _NCHUNK)], iidx_v)
        copies = []
        for j in range(_NCHUNK):
            dst = pl.ds(j * _CHUNK, _CHUNK)
            copies.append(pltpu.async_copy(
                utab_hbm.at[uidx_v.at[j]], urows_v.at[dst], usem))
            copies.append(pltpu.async_copy(
                itab_hbm.at[iidx_v.at[j]], irows_v.at[dst], isem))
        for c in copies:
            c.wait()
        pltpu.sync_copy(urows_v, uout_hbm.at[pl.ds(base, _BPW)])
        pltpu.sync_copy(irows_v, iout_hbm.at[pl.ds(base, _BPW)])

    return gather_kernel(user_idx2, item_idx2, user_table, item_table)


def _mlp_body(ie_ref, ue_ref, ai_ref, au_ref, b1_ref, w2_ref, b2_ref,
              w3_ref, b3_ref, out_ref):
    h = jnp.dot(ie_ref[...], ai_ref[...], preferred_element_type=jnp.float32)
    h = h + jnp.dot(ue_ref[...], au_ref[...], preferred_element_type=jnp.float32)
    h = jax.nn.relu(h + b1_ref[...])
    h = jax.nn.relu(
        jnp.dot(h, w2_ref[...], preferred_element_type=jnp.float32) + b2_ref[...])
    z = jnp.dot(h, w3_ref[...], preferred_element_type=jnp.float32)
    out_ref[...] = jax.nn.sigmoid(z[:, 0:1] + b3_ref[...])


def _tc_mlp(item_emb, user_emb, Ai, Au, b1p, W2p, b2p, W3p, b3p):
    bb = 2048
    grid = (BATCH // bb,)
    return pl.pallas_call(
        _mlp_body,
        grid=grid,
        in_specs=[
            pl.BlockSpec((bb, DIM), lambda i: (i, 0)),
            pl.BlockSpec((bb, DIM), lambda i: (i, 0)),
            pl.BlockSpec((DIM, HPAD), lambda i: (0, 0)),
            pl.BlockSpec((DIM, HPAD), lambda i: (0, 0)),
            pl.BlockSpec((1, HPAD), lambda i: (0, 0)),
            pl.BlockSpec((HPAD, HPAD), lambda i: (0, 0)),
            pl.BlockSpec((1, HPAD), lambda i: (0, 0)),
            pl.BlockSpec((HPAD, HPAD), lambda i: (0, 0)),
            pl.BlockSpec((1, 1), lambda i: (0, 0)),
        ],
        out_specs=pl.BlockSpec((bb, 1), lambda i: (i, 0)),
        out_shape=jax.ShapeDtypeStruct((BATCH, 1), jnp.float32),
    )(item_emb, user_emb, Ai, Au, b1p, W2p, b2p, W3p, b3p)


def kernel(user, item, user_table, item_table, gamma, beta, mean, var,
           W1, b1, W2, b2, W3, b3):
    s = gamma * lax.rsqrt(var + EPS)
    t = beta - mean * s
    A = s[:, None] * W1
    b1f = b1 + t @ W1
    Ai = jnp.zeros((DIM, HPAD), jnp.float32).at[:, :50].set(A[:DIM])
    Au = jnp.zeros((DIM, HPAD), jnp.float32).at[:, :50].set(A[DIM:])
    b1p = jnp.zeros((1, HPAD), jnp.float32).at[0, :50].set(b1f)
    W2p = jnp.zeros((HPAD, HPAD), jnp.float32).at[:50, :50].set(W2)
    b2p = jnp.zeros((1, HPAD), jnp.float32).at[0, :50].set(b2)
    W3p = jnp.zeros((HPAD, HPAD), jnp.float32).at[:50, 0:1].set(W3)
    b3p = b3.reshape(1, 1)

    user_emb, item_emb = _sc_gather(
        user.reshape(BATCH // _CHUNK, _CHUNK),
        item.reshape(BATCH // _CHUNK, _CHUNK),
        user_table, item_table)
    return _tc_mlp(item_emb, user_emb, Ai, Au, b1p, W2p, b2p, W3p, b3p)


# speedup vs baseline: 3.6385x; 3.6385x over previous
import functools

import jax
import jax.numpy as jnp
from jax import lax
from jax.experimental import pallas as pl
from jax.experimental.pallas import tpu as pltpu
from jax.experimental.pallas import tpu_sc as plsc

BATCH = 16384
DIM = 32
HPAD = 64
EPS = 1e-3

_NC = 2
_NS = 16
_NW = _NC * _NS
_BPW = BATCH // _NW
_G = 16
_NG = _BPW // _G


def _sc_gather_t(user_idx, item_idx, utab_t, itab_t):
    mesh = plsc.VectorSubcoreMesh(core_axis_name="c", subcore_axis_name="s")

    @functools.partial(
        pl.kernel,
        mesh=mesh,
        compiler_params=pltpu.CompilerParams(
            use_tc_tiling_on_sc=True, needs_layout_passes=False),
        out_type=[
            jax.ShapeDtypeStruct((DIM, BATCH), jnp.float32),
            jax.ShapeDtypeStruct((DIM, BATCH), jnp.float32),
        ],
        scratch_types=[
            pltpu.VMEM((_BPW,), jnp.int32),
            pltpu.VMEM((_BPW,), jnp.int32),
            pltpu.VMEM((_G * DIM, 128), jnp.float32),
            pltpu.VMEM((DIM, _BPW), jnp.float32),
            pltpu.SemaphoreType.DMA,
        ],
    )
    def gather_kernel(uidx_hbm, iidx_hbm, utab_hbm, itab_hbm,
                      uout_hbm, iout_hbm,
                      uidx_v, iidx_v, slab_v, rows_v, sem):
        wid = lax.axis_index("s") * _NC + lax.axis_index("c")
        base = wid * _BPW
        pltpu.sync_copy(uidx_hbm.at[pl.ds(base, _BPW)], uidx_v)
        pltpu.sync_copy(iidx_hbm.at[pl.ds(base, _BPW)], iidx_v)
        lanes = lax.iota(jnp.int32, 16)

        def one_table(idx_v, tab_hbm, out_hbm):
            def grp(g, _):
                chunk = idx_v[pl.ds(g * _G, _G)]

                def fire(k, _):
                    r = lax.reduce_max(
                        jnp.where(lanes == k, chunk, 0), (0,))
                    c0 = pl.multiple_of((r // 128) * 128, 128)
                    pltpu.async_copy(
                        tab_hbm.at[:, pl.ds(c0, 128)],
                        slab_v.at[pl.ds(k * DIM, DIM)], sem)
                    return 0

                lax.fori_loop(0, _G, fire, 0)

                def drain(k, _):
                    pltpu.make_async_copy(
                        tab_hbm.at[:, pl.ds(0, 128)],
                        slab_v.at[pl.ds(0, DIM)], sem).wait()
                    return 0

                lax.fori_loop(0, _G, drain, 0)
                lv = jnp.bitwise_and(chunk, 127)
                for f in range(DIM):
                    vals = plsc.load_gather(
                        slab_v, [lanes * DIM + f, lv])
                    rows_v[f, pl.ds(g * _G, _G)] = vals
                return 0

            lax.fori_loop(0, _NG, grp, 0)
            pltpu.sync_copy(rows_v, out_hbm.at[:, pl.ds(base, _BPW)])

        one_table(uidx_v, utab_hbm, uout_hbm)
        one_table(iidx_v, itab_hbm, iout_hbm)

    return gather_kernel(user_idx, item_idx, utab_t, itab_t)


def _mlp_body(ie_ref, ue_ref, ai_ref, au_ref, b1_ref, w2_ref, b2_ref,
              w3_ref, b3_ref, out_ref):
    h = jnp.dot(ai_ref[...], ie_ref[...], preferred_element_type=jnp.float32)
    h = h + jnp.dot(au_ref[...], ue_ref[...], preferred_element_type=jnp.float32)
    h = jax.nn.relu(h + b1_ref[...])
    h = jax.nn.relu(
        jnp.dot(w2_ref[...], h, preferred_element_type=jnp.float32) + b2_ref[...])
    z = jnp.dot(w3_ref[...], h, preferred_element_type=jnp.float32)
    out_ref[...] = jax.nn.sigmoid(z[0:1, :] + b3_ref[...])


def _tc_mlp(i_embT, u_embT, AiT, AuT, b1c, W2T, b2c, W3T, b3c):
    bb = 2048
    grid = (BATCH // bb,)
    return pl.pallas_call(
        _mlp_body,
        grid=grid,
        in_specs=[
            pl.BlockSpec((DIM, bb), lambda i: (0, i)),
            pl.BlockSpec((DIM, bb), lambda i: (0, i)),
            pl.BlockSpec((HPAD, DIM), lambda i: (0, 0)),
            pl.BlockSpec((HPAD, DIM), lambda i: (0, 0)),
            pl.BlockSpec((HPAD, 1), lambda i: (0, 0)),
            pl.BlockSpec((HPAD, HPAD), lambda i: (0, 0)),
            pl.BlockSpec((HPAD, 1), lambda i: (0, 0)),
            pl.BlockSpec((8, HPAD), lambda i: (0, 0)),
            pl.BlockSpec((1, 1), lambda i: (0, 0)),
        ],
        out_specs=pl.BlockSpec((1, bb), lambda i: (0, i)),
        out_shape=jax.ShapeDtypeStruct((1, BATCH), jnp.float32),
    )(i_embT, u_embT, AiT, AuT, b1c, W2T, b2c, W3T, b3c)


def kernel(user, item, user_table, item_table, gamma, beta, mean, var,
           W1, b1, W2, b2, W3, b3):
    s = gamma * lax.rsqrt(var + EPS)
    t = beta - mean * s
    A = s[:, None] * W1
    b1f = b1 + t @ W1
    AiT = jnp.zeros((HPAD, DIM), jnp.float32).at[:50].set(A[:DIM].T)
    AuT = jnp.zeros((HPAD, DIM), jnp.float32).at[:50].set(A[DIM:].T)
    b1c = jnp.zeros((HPAD, 1), jnp.float32).at[:50, 0].set(b1f)
    W2T = jnp.zeros((HPAD, HPAD), jnp.float32).at[:50, :50].set(W2.T)
    b2c = jnp.zeros((HPAD, 1), jnp.float32).at[:50, 0].set(b2)
    W3T = jnp.zeros((8, HPAD), jnp.float32).at[0, :50].set(W3[:, 0])
    b3c = b3.reshape(1, 1)

    u_embT, i_embT = _sc_gather_t(user, item, user_table.T, item_table.T)
    out_t = _tc_mlp(i_embT, u_embT, AiT, AuT, b1c, W2T, b2c, W3T, b3c)
    return out_t.reshape(BATCH, 1)
